# baseline (speedup 1.0000x reference)
import jax
import jax.numpy as jnp
from jax import lax
from jax.experimental import pallas as pl
from jax.experimental.pallas import tpu as pltpu
from jax.experimental.pallas import tpu_sc as plsc

_NROWS = 32768
_D = 1024
_B = 16
_SEGS_PER_CORE = _B // 2
_NSUB = 16
_R = 32
_NG = _D // 16


def _sc_body(x_hbm, ends_hbm, o_hbm,
             ends_v, buf0, buf1, acc_v, mbuf, shared, sem0, sem1):
    c = lax.axis_index("c")
    s = lax.axis_index("s")

    pltpu.sync_copy(ends_hbm, ends_v)
    evs = ends_v[...]
    e = [evs[k] for k in range(_B)]

    cstart = jnp.where(c == 0, jnp.int32(0), e[_SEGS_PER_CORE - 1])
    cend = jnp.where(c == 0, e[_SEGS_PER_CORE - 1], e[_B - 1])
    total = cend - cstart
    q = total // _NSUB
    r = total % _NSUB
    my_lo = cstart + s * q + jnp.minimum(s, r)
    my_hi = my_lo + q + jnp.where(s < r, 1, 0)

    neg = jnp.full((16,), -jnp.inf, jnp.float32)
    bufs = (buf0, buf1)
    sems = (sem0, sem1)

    def fold_rows(load, j_lo, j_hi):
        npairs = (j_hi - j_lo) // 2
        tail_lo = j_lo + 2 * npairs
        for gh in range(4):
            base_g = gh * 16
            accs = tuple(
                acc_v[pl.ds((base_g + g) * 16, 16)] for g in range(16))

            def pair_body(p, a):
                j = j_lo + 2 * p
                return tuple(
                    jnp.maximum(a[g],
                                jnp.maximum(load(j, (base_g + g) * 16),
                                            load(j + 1, (base_g + g) * 16)))
                    for g in range(16))

            def row_body(j, a):
                return tuple(
                    jnp.maximum(a[g], load(j, (base_g + g) * 16))
                    for g in range(16))

            accs = lax.fori_loop(0, npairs, pair_body, accs)
            accs = lax.fori_loop(tail_lo, j_hi, row_body, accs)
            for g in range(16):
                acc_v[pl.ds((base_g + g) * 16, 16)] = accs[g]

    def seg_body(k, carry):
        kg = c * _SEGS_PER_CORE + k
        sstart = jnp.int32(0)
        send = jnp.int32(0)
        for j in range(_B):
            send = jnp.where(kg == j, e[j], send)
            sstart = jnp.where(kg == j + 1, e[j], sstart)

        lo = jnp.maximum(my_lo, sstart)
        hi = jnp.minimum(my_hi, send)

        for g in range(_NG):
            acc_v[pl.ds(g * 16, 16)] = neg

        lo8 = (lo // 8) * 8
        nchunks = jnp.where(hi > lo, (hi - lo8 + _R - 1) // _R, 0)

        def chunk_st(kc):
            return jnp.minimum(lo8 + kc * _R, _NROWS - _R)

        def issue(kc, b):
            pltpu.make_async_copy(
                x_hbm.at[pl.ds(chunk_st(kc), _R)], bufs[b], sems[b]).start()

        def drain(b):
            pltpu.make_async_copy(
                x_hbm.at[pl.ds(0, _R)], bufs[b], sems[b]).wait()

        def process(kc, b):
            st = chunk_st(kc)
            j_lo = jnp.maximum(0, lo - st)
            j_hi = jnp.minimum(_R, hi - st)
            buf = bufs[b]
            fold_rows(lambda j, off: buf[j, pl.ds(off, 16)], j_lo, j_hi)

        @pl.when(nchunks > 0)
        def _prime():
            issue(0, 0)

        def pair(p, cr):
            k0 = 2 * p

            @pl.when(k0 + 1 < nchunks)
            def _():
                issue(k0 + 1, 1)

            drain(0)
            process(k0, 0)

            @pl.when(k0 + 2 < nchunks)
            def _():
                issue(k0 + 2, 0)

            @pl.when(k0 + 1 < nchunks)
            def _():
                drain(1)
                process(k0 + 1, 1)

            return cr

        lax.fori_loop(0, (nchunks + 1) // 2, pair, 0)

        pltpu.sync_copy(acc_v, shared.at[pl.ds((k * _NSUB + s) * _D, _D)])
        return carry

    lax.fori_loop(0, _SEGS_PER_CORE, seg_body, 0)
    plsc.subcore_barrier()

    @pl.when(s < _SEGS_PER_CORE)
    def _merge():
        pltpu.sync_copy(
            shared.at[pl.ds(s * _NSUB * _D, _NSUB * _D)], mbuf)
        for g in range(_NG):
            acc_v[pl.ds(g * 16, 16)] = mbuf[pl.ds(g * 16, 16)]
        fold_rows(lambda j, off: mbuf[pl.ds(j * _D + off, 16)], 1, _NSUB)
        kg = c * _SEGS_PER_CORE + s
        pltpu.sync_copy(acc_v, o_hbm.at[pl.ds(kg * _D, _D)])


def kernel(input, sizes):
    ends32 = jnp.cumsum(sizes.astype(jnp.int32))
    mesh = plsc.VectorSubcoreMesh(
        core_axis_name="c", subcore_axis_name="s",
        num_cores=2, num_subcores=16)
    f = pl.kernel(
        _sc_body,
        out_type=jax.ShapeDtypeStruct((_B * _D,), jnp.float32),
        mesh=mesh,
        scratch_types=[
            pltpu.VMEM((_B,), jnp.int32),
            pltpu.VMEM((_R, _D), jnp.float32),
            pltpu.VMEM((_R, _D), jnp.float32),
            pltpu.VMEM((_D,), jnp.float32),
            pltpu.VMEM((_NSUB * _D,), jnp.float32),
            pltpu.VMEM_SHARED((_SEGS_PER_CORE * _NSUB * _D,),
                              jnp.float32),
            pltpu.SemaphoreType.DMA,
            pltpu.SemaphoreType.DMA,
        ],
    )
    return f(input, ends32).reshape(_B, _D)


# speedup vs baseline: 1.0134x; 1.0134x over previous
import jax
import jax.numpy as jnp
from jax import lax
from jax.experimental import pallas as pl
from jax.experimental.pallas import tpu as pltpu
from jax.experimental.pallas import tpu_sc as plsc

_NROWS = 32768
_D = 1024
_B = 16
_SEGS_PER_CORE = _B // 2
_NSUB = 16
_R = 48
_NG = _D // 16


def _sc_body(x_hbm, ends_hbm, o_hbm,
             ends_v, buf0, buf1, acc_v, mbuf, shared, sem0, sem1):
    c = lax.axis_index("c")
    s = lax.axis_index("s")

    pltpu.sync_copy(ends_hbm, ends_v)
    evs = ends_v[...]
    e = [evs[k] for k in range(_B)]

    cstart = jnp.where(c == 0, jnp.int32(0), e[_SEGS_PER_CORE - 1])
    cend = jnp.where(c == 0, e[_SEGS_PER_CORE - 1], e[_B - 1])
    total = cend - cstart
    q = total // _NSUB
    r = total % _NSUB
    my_lo = cstart + s * q + jnp.minimum(s, r)
    my_hi = my_lo + q + jnp.where(s < r, 1, 0)

    neg = jnp.full((16,), -jnp.inf, jnp.float32)
    bufs = (buf0, buf1)
    sems = (sem0, sem1)

    def fold_rows(load, j_lo, j_hi):
        for gh in range(4):
            base_g = gh * 16
            accs = tuple(
                acc_v[pl.ds((base_g + g) * 16, 16)] for g in range(16))

            def row_body(j, a):
                return tuple(
                    jnp.maximum(a[g], load(j, (base_g + g) * 16))
                    for g in range(16))

            accs = lax.fori_loop(j_lo, j_hi, row_body, accs)
            for g in range(16):
                acc_v[pl.ds((base_g + g) * 16, 16)] = accs[g]

    def seg_body(k, carry):
        kg = c * _SEGS_PER_CORE + k
        sstart = jnp.int32(0)
        send = jnp.int32(0)
        for j in range(_B):
            send = jnp.where(kg == j, e[j], send)
            sstart = jnp.where(kg == j + 1, e[j], sstart)

        lo = jnp.maximum(my_lo, sstart)
        hi = jnp.minimum(my_hi, send)

        for g in range(_NG):
            acc_v[pl.ds(g * 16, 16)] = neg

        lo8 = (lo // 8) * 8
        nchunks = jnp.where(hi > lo, (hi - lo8 + _R - 1) // _R, 0)

        def chunk_st(kc):
            return jnp.minimum(lo8 + kc * _R, _NROWS - _R)

        def issue(kc, b):
            pltpu.make_async_copy(
                x_hbm.at[pl.ds(chunk_st(kc), _R)], bufs[b], sems[b]).start()

        def drain(b):
            pltpu.make_async_copy(
                x_hbm.at[pl.ds(0, _R)], bufs[b], sems[b]).wait()

        def process(kc, b):
            st = chunk_st(kc)
            j_lo = jnp.maximum(0, lo - st)
            j_hi = jnp.minimum(_R, hi - st)
            buf = bufs[b]
            fold_rows(lambda j, off: buf[j, pl.ds(off, 16)], j_lo, j_hi)

        @pl.when(nchunks > 0)
        def _prime():
            issue(0, 0)

        def pair(p, cr):
            k0 = 2 * p

            @pl.when(k0 + 1 < nchunks)
            def _():
                issue(k0 + 1, 1)

            drain(0)
            process(k0, 0)

            @pl.when(k0 + 2 < nchunks)
            def _():
                issue(k0 + 2, 0)

            @pl.when(k0 + 1 < nchunks)
            def _():
                drain(1)
                process(k0 + 1, 1)

            return cr

        lax.fori_loop(0, (nchunks + 1) // 2, pair, 0)

        pltpu.sync_copy(acc_v, shared.at[pl.ds((k * _NSUB + s) * _D, _D)])
        return carry

    lax.fori_loop(0, _SEGS_PER_CORE, seg_body, 0)
    plsc.subcore_barrier()

    @pl.when(s < _SEGS_PER_CORE)
    def _merge():
        pltpu.sync_copy(
            shared.at[pl.ds(s * _NSUB * _D, _NSUB * _D)], mbuf)
        for g in range(_NG):
            acc_v[pl.ds(g * 16, 16)] = mbuf[pl.ds(g * 16, 16)]
        fold_rows(lambda j, off: mbuf[pl.ds(j * _D + off, 16)], 1, _NSUB)
        kg = c * _SEGS_PER_CORE + s
        pltpu.sync_copy(acc_v, o_hbm.at[pl.ds(kg * _D, _D)])


def kernel(input, sizes):
    ends32 = jnp.cumsum(sizes.astype(jnp.int32))
    mesh = plsc.VectorSubcoreMesh(
        core_axis_name="c", subcore_axis_name="s",
        num_cores=2, num_subcores=16)
    f = pl.kernel(
        _sc_body,
        out_type=jax.ShapeDtypeStruct((_B * _D,), jnp.float32),
        mesh=mesh,
        scratch_types=[
            pltpu.VMEM((_B,), jnp.int32),
            pltpu.VMEM((_R, _D), jnp.float32),
            pltpu.VMEM((_R, _D), jnp.float32),
            pltpu.VMEM((_D,), jnp.float32),
            pltpu.VMEM((_NSUB * _D,), jnp.float32),
            pltpu.VMEM_SHARED((_SEGS_PER_CORE * _NSUB * _D,),
                              jnp.float32),
            pltpu.SemaphoreType.DMA,
            pltpu.SemaphoreType.DMA,
        ],
    )
    return f(input, ends32).reshape(_B, _D)


# speedup vs baseline: 1.0356x; 1.0219x over previous
import jax
import jax.numpy as jnp
from jax import lax
from jax.experimental import pallas as pl
from jax.experimental.pallas import tpu as pltpu
from jax.experimental.pallas import tpu_sc as plsc

_NROWS = 32768
_D = 1024
_B = 16
_SEGS_PER_CORE = _B // 2
_NSUB = 16
_R = 32
_NG = _D // 16


def _sc_body(x_hbm, ends_hbm, o_hbm,
             ends_v, buf0, buf1, acc_v, mbuf, shared, sem0, sem1):
    c = lax.axis_index("c")
    s = lax.axis_index("s")

    pltpu.sync_copy(ends_hbm, ends_v)
    evs = ends_v[...]
    e = [evs[k] for k in range(_B)]

    cstart = jnp.where(c == 0, jnp.int32(0), e[_SEGS_PER_CORE - 1])
    cend = jnp.where(c == 0, e[_SEGS_PER_CORE - 1], e[_B - 1])
    total = cend - cstart
    q = total // _NSUB
    r = total % _NSUB
    my_lo = cstart + s * q + jnp.minimum(s, r)
    my_hi = my_lo + q + jnp.where(s < r, 1, 0)

    neg = jnp.full((16,), -jnp.inf, jnp.float32)
    bufs = (buf0, buf1)
    sems = (sem0, sem1)

    def fold_rows(load, j_lo, j_hi):
        for gh in range(4):
            base_g = gh * 16
            accs = tuple(
                acc_v[pl.ds((base_g + g) * 16, 16)] for g in range(16))

            def row_body(j, a):
                return tuple(
                    jnp.maximum(a[g], load(j, (base_g + g) * 16))
                    for g in range(16))

            accs = lax.fori_loop(j_lo, j_hi, row_body, accs)
            for g in range(16):
                acc_v[pl.ds((base_g + g) * 16, 16)] = accs[g]

    def seg_body(k, carry):
        kg = c * _SEGS_PER_CORE + k
        sstart = jnp.int32(0)
        send = jnp.int32(0)
        for j in range(_B):
            send = jnp.where(kg == j, e[j], send)
            sstart = jnp.where(kg == j + 1, e[j], sstart)

        lo = jnp.maximum(my_lo, sstart)
        hi = jnp.minimum(my_hi, send)

        for g in range(_NG):
            acc_v[pl.ds(g * 16, 16)] = neg

        lo8 = (lo // 8) * 8
        nchunks = jnp.where(hi > lo, (hi - lo8 + _R - 1) // _R, 0)

        def chunk_st(kc):
            return jnp.minimum(lo8 + kc * _R, _NROWS - _R)

        def issue(kc, b):
            pltpu.make_async_copy(
                x_hbm.at[pl.ds(chunk_st(kc), _R)], bufs[b], sems[b]).start()

        def drain(b):
            pltpu.make_async_copy(
                x_hbm.at[pl.ds(0, _R)], bufs[b], sems[b]).wait()

        def process(kc, b):
            st = chunk_st(kc)
            j_lo = jnp.maximum(0, lo - st)
            j_hi = jnp.minimum(_R, hi - st)
            buf = bufs[b]
            fold_rows(lambda j, off: buf[j, pl.ds(off, 16)], j_lo, j_hi)

        @pl.when(nchunks > 0)
        def _prime():
            issue(0, 0)

        def pair(p, cr):
            k0 = 2 * p

            @pl.when(k0 + 1 < nchunks)
            def _():
                issue(k0 + 1, 1)

            drain(0)
            process(k0, 0)

            @pl.when(k0 + 2 < nchunks)
            def _():
                issue(k0 + 2, 0)

            @pl.when(k0 + 1 < nchunks)
            def _():
                drain(1)
                process(k0 + 1, 1)

            return cr

        lax.fori_loop(0, (nchunks + 1) // 2, pair, 0)

        pltpu.sync_copy(acc_v, shared.at[pl.ds((k * _NSUB + s) * _D, _D)])
        return carry

    lax.fori_loop(0, _SEGS_PER_CORE, seg_body, 0)
    plsc.subcore_barrier()

    @pl.when(s < _SEGS_PER_CORE)
    def _merge():
        pltpu.sync_copy(
            shared.at[pl.ds(s * _NSUB * _D, _NSUB * _D)], mbuf)
        for g in range(_NG):
            acc_v[pl.ds(g * 16, 16)] = mbuf[pl.ds(g * 16, 16)]
        fold_rows(lambda j, off: mbuf[pl.ds(j * _D + off, 16)], 1, _NSUB)
        kg = c * _SEGS_PER_CORE + s
        pltpu.sync_copy(acc_v, o_hbm.at[pl.ds(kg * _D, _D)])


def kernel(input, sizes):
    ends32 = jnp.cumsum(sizes.astype(jnp.int32))
    mesh = plsc.VectorSubcoreMesh(
        core_axis_name="c", subcore_axis_name="s",
        num_cores=2, num_subcores=16)
    f = pl.kernel(
        _sc_body,
        out_type=jax.ShapeDtypeStruct((_B * _D,), jnp.float32),
        mesh=mesh,
        scratch_types=[
            pltpu.VMEM((_B,), jnp.int32),
            pltpu.VMEM((_R, _D), jnp.float32),
            pltpu.VMEM((_R, _D), jnp.float32),
            pltpu.VMEM((_D,), jnp.float32),
            pltpu.VMEM((_NSUB * _D,), jnp.float32),
            pltpu.VMEM_SHARED((_SEGS_PER_CORE * _NSUB * _D,),
                              jnp.float32),
            pltpu.SemaphoreType.DMA,
            pltpu.SemaphoreType.DMA,
        ],
    )
    return f(input, ends32).reshape(_B, _D)
